# baseline (speedup 1.0000x reference)
import jax
import jax.numpy as jnp
from jax import lax
from jax.experimental import pallas as pl
from jax.experimental.pallas import tpu as pltpu
from jax.experimental.pallas import tpu_sc as plsc

N = 32768
D = 768
E = 8
L = 16
NW = 32

N_SC = 8192
N_TC = N - N_SC

BLK = 2048
NBLK = N_TC // BLK
NBUF = 4


def _tc_body(x_hbm, w_ref, b_ref, o_ref, *scr):
    xbufs = scr[:NBUF]
    sems = scr[NBUF:]

    def src(i):
        return x_hbm.at[pl.ds(i * BLK, BLK), :]

    for i in range(min(NBUF, NBLK)):
        pltpu.make_async_copy(src(i), xbufs[i], sems[i]).start()

    for i in range(NBLK):
        bi = i % NBUF
        pltpu.make_async_copy(src(i), xbufs[bi], sems[bi]).wait()
        logits = jnp.dot(
            xbufs[bi][...], w_ref[...], preferred_element_type=jnp.float32
        ) + b_ref[...]
        m = jnp.max(logits, axis=-1, keepdims=True)
        ex = jnp.exp(logits - m)
        o_ref[pl.ds(i * BLK, BLK), :] = ex / jnp.sum(ex, axis=-1, keepdims=True)
        if i + NBUF < NBLK:
            pltpu.make_async_copy(src(i + NBUF), xbufs[bi], sems[bi]).start()


def _tc_router(x_tc, Wt, b2):
    return pl.pallas_call(
        _tc_body,
        in_specs=[
            pl.BlockSpec(memory_space=pltpu.MemorySpace.HBM),
            pl.BlockSpec(memory_space=pltpu.VMEM),
            pl.BlockSpec(memory_space=pltpu.VMEM),
        ],
        out_specs=pl.BlockSpec(memory_space=pltpu.VMEM),
        out_shape=jax.ShapeDtypeStruct((N, E), jnp.float32),
        scratch_shapes=(
            [pltpu.VMEM((BLK, D), jnp.float32) for _ in range(NBUF)]
            + [pltpu.SemaphoreType.DMA for _ in range(NBUF)]
        ),
    )(x_tc, Wt, b2)


CHUNK = 64
TG = 4
DC = D // L

_GATHER_DNUMS = lax.GatherDimensionNumbers(
    offset_dims=(), collapsed_slice_dims=(0,), start_index_map=(0,)
)


def _permute(v, idx):
    return lax.gather(
        v, idx[:, None], _GATHER_DNUMS, slice_sizes=(1,),
        mode=lax.GatherScatterMode.PROMISE_IN_BOUNDS,
    )


def _seg8(v, op, idx):
    for s in (4, 2, 1):
        v = op(v, _permute(v, idx ^ s))
    return v


def _transpose_reduce16(vecs, idx, masks):
    for s, m in zip((1, 2, 4, 8), masks):
        nxt = []
        for k in range(len(vecs) // 2):
            a, b = vecs[2 * k], vecs[2 * k + 1]
            pa = _permute(a, idx ^ s)
            pb = _permute(b, idx ^ s)
            nxt.append(jnp.where(m, a, pb) + jnp.where(m, pa, b))
        vecs = nxt
    return vecs[0]


def _make_sc_body(tpw, tok_offset):
    nchunk = tpw // CHUNK

    def _sc_body(x_hbm, b16_hbm, w_hbm, out_hbm, w_v, b_v, x_v0, x_v1, o_v,
                 sem0, sem1):
        wid = lax.axis_index("s") * 2 + lax.axis_index("c")
        base = tok_offset + wid * tpw

        pltpu.sync_copy(w_hbm, w_v)
        pltpu.sync_copy(b16_hbm, b_v)
        b16 = b_v[...]
        idx = lax.iota(jnp.int32, L)
        masks = [(idx & s) == 0 for s in (1, 2, 4, 8)]
        bufs = (x_v0, x_v1)
        sems = (sem0, sem1)

        def src(ci):
            row = pl.multiple_of(base + ci * CHUNK, 8)
            return x_hbm.at[pl.ds(row, CHUNK), :]

        def compute(ci, x_v):
            tok0 = ci * CHUNK

            def tg_body(g, _):
                t0 = g * TG

                def c_body(c, accs):
                    col = c * L
                    xs = [x_v[t0 + t, pl.ds(col, L)] for t in range(TG)]
                    ws = [w_v[e, pl.ds(col, L)] for e in range(E)]
                    return tuple(
                        accs[t * E + e] + xs[t] * ws[e]
                        for t in range(TG)
                        for e in range(E)
                    )

                accs = lax.fori_loop(
                    0, DC, c_body,
                    tuple(jnp.zeros((L,), jnp.float32) for _ in range(TG * E)),
                )

                for pair in range(TG // 2):
                    v = _transpose_reduce16(
                        list(accs[pair * 2 * E:(pair * 2 + 2) * E]), idx, masks
                    ) + b16
                    m = _seg8(v, jnp.maximum, idx)
                    ex = jnp.exp(v - m)
                    s = _seg8(ex, jnp.add, idx)
                    o_v[pl.ds((tok0 + t0 + 2 * pair) * E, 2 * E)] = ex / s
                return 0

            lax.fori_loop(0, CHUNK // TG, tg_body, 0)

        pltpu.async_copy(src(0), bufs[0], sems[0])
        if nchunk > 1:
            pltpu.async_copy(src(1), bufs[1], sems[1])
        for ci in range(nchunk):
            bi = ci % 2
            pltpu.make_async_copy(src(ci), bufs[bi], sems[bi]).wait()
            compute(ci, bufs[bi])
            if ci + 2 < nchunk:
                pltpu.async_copy(src(ci + 2), bufs[bi], sems[bi])

        ooff = pl.multiple_of((base - tok_offset) * E, 8)
        pltpu.sync_copy(o_v, out_hbm.at[pl.ds(ooff, tpw * E)])

    return _sc_body


def _sc_router(x1, b16, w1, n_tok, tok_offset):
    tpw = n_tok // NW
    mesh = plsc.VectorSubcoreMesh(core_axis_name="c", subcore_axis_name="s")
    return pl.kernel(
        _make_sc_body(tpw, tok_offset),
        mesh=mesh,
        out_type=jax.ShapeDtypeStruct((n_tok * E,), jnp.float32),
        scratch_types=[
            pltpu.VMEM((E, D), jnp.float32),
            pltpu.VMEM((L,), jnp.float32),
            pltpu.VMEM((CHUNK, D), jnp.float32),
            pltpu.VMEM((CHUNK, D), jnp.float32),
            pltpu.VMEM((tpw * E,), jnp.float32),
            pltpu.SemaphoreType.DMA,
            pltpu.SemaphoreType.DMA,
        ],
    )(x1, b16, w1)


def kernel(x, W, b):
    Wt = W.T
    b2 = b.reshape(1, E)
    b16 = jnp.tile(b, 2)
    out_sc = _sc_router(x, b16, W, N_SC, N_TC).reshape(N_SC, E)
    out_tc = _tc_router(x, Wt, b2)
    return lax.dynamic_update_slice(out_tc, out_sc, (N_TC, 0))


# speedup vs baseline: 1.1487x; 1.1487x over previous
import jax
import jax.numpy as jnp
from jax.experimental import pallas as pl
from jax.experimental.pallas import tpu as pltpu

N = 32768
D = 768
E = 8
BLK = 512
NBLK = N // BLK
NBUF = 16


def _body(x_hbm, w_ref, b_ref, o_ref, *scr):
    xbufs = scr[:NBUF]
    sems = scr[NBUF:]

    def src(i):
        return x_hbm.at[pl.ds(i * BLK, BLK), :]

    for i in range(min(NBUF, NBLK)):
        pltpu.make_async_copy(src(i), xbufs[i], sems[i]).start()

    for i in range(NBLK):
        bi = i % NBUF
        pltpu.make_async_copy(src(i), xbufs[bi], sems[bi]).wait()
        logits = jnp.dot(
            xbufs[bi][...], w_ref[...], preferred_element_type=jnp.float32
        ) + b_ref[...]
        m = jnp.max(logits, axis=-1, keepdims=True)
        ex = jnp.exp(logits - m)
        o_ref[pl.ds(i * BLK, BLK), :] = ex / jnp.sum(ex, axis=-1, keepdims=True)
        if i + NBUF < NBLK:
            pltpu.make_async_copy(src(i + NBUF), xbufs[bi], sems[bi]).start()


def kernel(x, W, b):
    Wt = W.T
    b2 = b.reshape(1, E)
    out = pl.pallas_call(
        _body,
        in_specs=[
            pl.BlockSpec(memory_space=pltpu.MemorySpace.HBM),
            pl.BlockSpec(memory_space=pltpu.VMEM),
            pl.BlockSpec(memory_space=pltpu.VMEM),
        ],
        out_specs=pl.BlockSpec(memory_space=pltpu.VMEM),
        out_shape=jax.ShapeDtypeStruct((N, E), jnp.float32),
        scratch_shapes=(
            [pltpu.VMEM((BLK, D), jnp.float32) for _ in range(NBUF)]
            + [pltpu.SemaphoreType.DMA for _ in range(NBUF)]
        ),
    )(x, Wt, b2)
    return out


# speedup vs baseline: 1.4620x; 1.2727x over previous
import jax
import jax.numpy as jnp
from jax.experimental import pallas as pl


def _router_block(x_ref, w_ref, b_ref, o_ref):
    logits = jnp.dot(x_ref[...], w_ref[...], preferred_element_type=jnp.float32)
    logits = logits + b_ref[...]
    m = jnp.max(logits, axis=-1, keepdims=True)
    e = jnp.exp(logits - m)
    o_ref[...] = e / jnp.sum(e, axis=-1, keepdims=True)


def kernel(x, W, b):
    N, D = x.shape
    E = W.shape[0]
    BLOCK = 4096
    Wt = W.T
    b2 = b.reshape(1, E)
    out = pl.pallas_call(
        _router_block,
        grid=(N // BLOCK,),
        in_specs=[
            pl.BlockSpec((BLOCK, D), lambda i: (i, 0)),
            pl.BlockSpec((D, E), lambda i: (0, 0)),
            pl.BlockSpec((1, E), lambda i: (0, 0)),
        ],
        out_specs=pl.BlockSpec((BLOCK, E), lambda i: (i, 0)),
        out_shape=jax.ShapeDtypeStruct((N, E), jnp.float32),
    )(x, Wt, b2)
    return out
